# baseline (speedup 1.0000x reference)
import functools

import jax
import jax.numpy as jnp
from jax.experimental import pallas as pl
from jax.experimental.pallas import tpu as pltpu

_NE = 1024
_D = 64
_R = 1024


def _vq_block(xt_ref, wm2_ref, wt_ref, icol_ref, q_ref, counts_ref, sse_ref):
    xt = xt_ref[0]
    wm2 = wm2_ref[...]
    x2 = jnp.sum(xt * xt, axis=0, keepdims=True)
    w2 = 0.25 * jnp.sum(wm2 * wm2, axis=1, keepdims=True)
    mm2 = jax.lax.dot_general(wm2, xt, (((1,), (0,)), ((), ())),
                              preferred_element_type=jnp.float32)
    d = (x2 + w2) + mm2
    icol = icol_ref[...]
    dmin = jnp.min(d, axis=0, keepdims=True)
    idx = jnp.min(jnp.where(d == dmin, icol, float(_NE)), axis=0,
                  keepdims=True)
    onehot_t = (icol == idx).astype(jnp.float32)
    qt = jax.lax.dot_general(wt_ref[...], onehot_t, (((1,), (0,)), ((), ())),
                             preferred_element_type=jnp.float32)
    q_ref[0] = qt
    cb = jnp.sum(onehot_t, axis=1, keepdims=True)
    sb = jnp.sum(dmin, axis=1, keepdims=True)

    @pl.when(pl.program_id(0) == 0)
    def _init():
        counts_ref[...] = cb
        sse_ref[...] = sb

    @pl.when(pl.program_id(0) != 0)
    def _acc():
        counts_ref[...] += cb
        sse_ref[...] += sb


@functools.partial(jax.jit, static_argnames=())
def kernel(x, W):
    B, C, H, Wd = x.shape
    n = B * H * Wd
    xt3 = x.reshape(B, C, H * Wd)
    wm2 = -2.0 * W
    wt = W.T
    icol = jnp.arange(_NE, dtype=jnp.float32).reshape(_NE, 1)
    grid = n // _R
    qc, counts, sse = pl.pallas_call(
        _vq_block,
        grid=(grid,),
        in_specs=[
            pl.BlockSpec((1, _D, _R), lambda i: (i, 0, 0)),
            pl.BlockSpec((_NE, _D), lambda i: (0, 0)),
            pl.BlockSpec((_D, _NE), lambda i: (0, 0)),
            pl.BlockSpec((_NE, 1), lambda i: (0, 0)),
        ],
        out_specs=[
            pl.BlockSpec((1, _D, _R), lambda i: (i, 0, 0)),
            pl.BlockSpec((_NE, 1), lambda i: (0, 0)),
            pl.BlockSpec((1, 1), lambda i: (0, 0)),
        ],
        out_shape=[
            jax.ShapeDtypeStruct((B, C, H * Wd), jnp.float32),
            jax.ShapeDtypeStruct((_NE, 1), jnp.float32),
            jax.ShapeDtypeStruct((1, 1), jnp.float32),
        ],
        compiler_params=pltpu.CompilerParams(
            dimension_semantics=("arbitrary",),
        ),
    )(xt3, wm2, wt, icol)
    quantized = qc.reshape(B, C, H, Wd)
    m = sse[0, 0] / (n * _D)
    loss = m + 0.25 * m
    avg_probs = counts[:, 0] / n
    perplexity = jnp.exp(-jnp.sum(avg_probs * jnp.log(avg_probs + 1e-10)))
    return (quantized, loss, perplexity)


# speedup vs baseline: 1.1306x; 1.1306x over previous
import functools

import jax
import jax.numpy as jnp
from jax.experimental import pallas as pl
from jax.experimental.pallas import tpu as pltpu

_NE = 1024
_D = 64
_R = 1024
_N = 16384


def _vq_block(x_ref, w_ref, q_ref, loss_ref, perp_ref,
              wm2_sc, w2_sc, counts_sc, sse_sc):
    i = pl.program_id(0)

    @pl.when(i == 0)
    def _prep():
        wm2_sc[...] = -2.0 * w_ref[...]
        t = wm2_sc[...]
        w2_sc[...] = 0.25 * jnp.sum(t * t, axis=1, keepdims=True)

    wm2 = wm2_sc[...]
    icol = jax.lax.broadcasted_iota(jnp.int32, (_NE, 1), 0).astype(jnp.float32)
    xt = x_ref[0]
    x2 = jnp.sum(xt * xt, axis=0, keepdims=True)
    mm2 = jax.lax.dot_general(wm2, xt, (((1,), (0,)), ((), ())),
                              preferred_element_type=jnp.float32)
    d = (x2 + w2_sc[...]) + mm2
    dmin = jnp.min(d, axis=0, keepdims=True)
    idx = jnp.min(jnp.where(d == dmin, icol, float(_NE)), axis=0,
                  keepdims=True)
    onehot_t = (icol == idx).astype(jnp.float32)
    qt = -0.5 * jax.lax.dot_general(wm2, onehot_t, (((0,), (0,)), ((), ())),
                                    preferred_element_type=jnp.float32)
    q_ref[0] = qt
    cb = jnp.sum(onehot_t, axis=1, keepdims=True)
    sb = jnp.sum(dmin, axis=1, keepdims=True)

    @pl.when(i == 0)
    def _init():
        counts_sc[...] = cb
        sse_sc[...] = sb

    @pl.when(i != 0)
    def _acc():
        counts_sc[...] += cb
        sse_sc[...] += sb

    @pl.when(i == pl.num_programs(0) - 1)
    def _fin():
        avg = counts_sc[...] * (1.0 / _N)
        ent = avg * jnp.log(avg + 1e-10)
        perp_ref[...] = jnp.exp(-jnp.sum(ent, axis=0, keepdims=True))
        m = sse_sc[0, 0] * (1.0 / (_N * _D))
        loss_ref[...] = jnp.full((1, 1), m + 0.25 * m, jnp.float32)


@functools.partial(jax.jit, static_argnames=())
def kernel(x, W):
    B, C, H, Wd = x.shape
    xt3 = x.reshape(B, C, H * Wd)
    grid = (B * H * Wd) // _R
    qc, loss, perp = pl.pallas_call(
        _vq_block,
        grid=(grid,),
        in_specs=[
            pl.BlockSpec((1, _D, _R), lambda i: (i, 0, 0)),
            pl.BlockSpec((_NE, _D), lambda i: (0, 0)),
        ],
        out_specs=[
            pl.BlockSpec((1, _D, _R), lambda i: (i, 0, 0)),
            pl.BlockSpec((1, 1), lambda i: (0, 0)),
            pl.BlockSpec((1, 1), lambda i: (0, 0)),
        ],
        out_shape=[
            jax.ShapeDtypeStruct((B, C, H * Wd), jnp.float32),
            jax.ShapeDtypeStruct((1, 1), jnp.float32),
            jax.ShapeDtypeStruct((1, 1), jnp.float32),
        ],
        scratch_shapes=[
            pltpu.VMEM((_NE, _D), jnp.float32),
            pltpu.VMEM((_NE, 1), jnp.float32),
            pltpu.VMEM((_NE, 1), jnp.float32),
            pltpu.VMEM((1, 1), jnp.float32),
        ],
        compiler_params=pltpu.CompilerParams(
            dimension_semantics=("arbitrary",),
        ),
    )(xt3, W)
    return (qc.reshape(B, C, H, Wd), loss[0, 0], perp[0, 0])


# speedup vs baseline: 1.5179x; 1.3426x over previous
import functools

import jax
import jax.numpy as jnp
from jax.experimental import pallas as pl
from jax.experimental.pallas import tpu as pltpu

_NE = 1024
_D = 64
_R = 2048
_N = 16384


def _vq_block(x_ref, wt_ref, q_ref, loss_ref, perp_ref,
              wtm2_sc, w_sc, w2_sc, irow_sc, counts_sc, sse_sc):
    i = pl.program_id(0)

    @pl.when(i == 0)
    def _prep():
        wtm2_sc[...] = -2.0 * wt_ref[...]
        t = wtm2_sc[...]
        w2_sc[...] = 0.25 * jnp.sum(t * t, axis=0, keepdims=True)
        w_sc[...] = jnp.transpose(wt_ref[...], (1, 0))
        irow_sc[...] = jax.lax.broadcasted_iota(
            jnp.int32, (1, _NE), 1).astype(jnp.float32)

    irow = irow_sc[...]
    xb = x_ref[...]
    x2 = jnp.sum(xb * xb, axis=1, keepdims=True)
    mm2 = jax.lax.dot_general(xb, wtm2_sc[...], (((1,), (0,)), ((), ())),
                              preferred_element_type=jnp.float32)
    d = (x2 + w2_sc[...]) + mm2
    dmin = jnp.min(d, axis=1, keepdims=True)
    idx = jnp.min(jnp.where(d == dmin, irow, float(_NE)), axis=1,
                  keepdims=True)
    onehot = (irow == idx).astype(jnp.float32)
    qb = jax.lax.dot_general(onehot, w_sc[...], (((1,), (0,)), ((), ())),
                             preferred_element_type=jnp.float32)
    q_ref[...] = qb
    cb = jnp.sum(onehot, axis=0, keepdims=True)
    sb = jnp.sum(dmin, axis=0, keepdims=True)

    @pl.when(i == 0)
    def _init():
        counts_sc[...] = cb
        sse_sc[...] = sb

    @pl.when(i != 0)
    def _acc():
        counts_sc[...] += cb
        sse_sc[...] += sb

    @pl.when(i == pl.num_programs(0) - 1)
    def _fin():
        avg = counts_sc[...] * (1.0 / _N)
        ent = avg * jnp.log(avg + 1e-10)
        perp_ref[...] = jnp.exp(-jnp.sum(ent, axis=1, keepdims=True))
        m = sse_sc[0, 0] * (1.0 / (_N * _D))
        loss_ref[...] = jnp.full((1, 1), m + 0.25 * m, jnp.float32)


@functools.partial(jax.jit, static_argnames=())
def kernel(x, W):
    B, C, H, Wd = x.shape
    n = B * H * Wd
    x_flat = jnp.transpose(x, (0, 2, 3, 1)).reshape(n, _D)
    wt = W.T
    grid = n // _R
    q, loss, perp = pl.pallas_call(
        _vq_block,
        grid=(grid,),
        in_specs=[
            pl.BlockSpec((_R, _D), lambda i: (i, 0)),
            pl.BlockSpec((_D, _NE), lambda i: (0, 0)),
        ],
        out_specs=[
            pl.BlockSpec((_R, _D), lambda i: (i, 0)),
            pl.BlockSpec((1, 1), lambda i: (0, 0)),
            pl.BlockSpec((1, 1), lambda i: (0, 0)),
        ],
        out_shape=[
            jax.ShapeDtypeStruct((n, _D), jnp.float32),
            jax.ShapeDtypeStruct((1, 1), jnp.float32),
            jax.ShapeDtypeStruct((1, 1), jnp.float32),
        ],
        scratch_shapes=[
            pltpu.VMEM((_D, _NE), jnp.float32),
            pltpu.VMEM((_NE, _D), jnp.float32),
            pltpu.VMEM((1, _NE), jnp.float32),
            pltpu.VMEM((1, _NE), jnp.float32),
            pltpu.VMEM((1, _NE), jnp.float32),
            pltpu.VMEM((1, 1), jnp.float32),
        ],
        compiler_params=pltpu.CompilerParams(
            dimension_semantics=("arbitrary",),
        ),
    )(x_flat, wt)
    quantized = q.reshape(B, H, Wd, C).transpose(0, 3, 1, 2)
    return (quantized, loss[0, 0], perp[0, 0])
